# baseline (speedup 1.0000x reference)
import jax
import jax.numpy as jnp
from jax import lax
from jax.experimental import pallas as pl
from jax.experimental.pallas import tpu as pltpu
from jax.experimental.pallas import tpu_sc as plsc


_NC = 2
_NS = 16
_NW = _NC * _NS
_CHUNK_ROWS = 32


def _sc_copy_body(w_hbm, o_hbm, buf0, buf1, lsem0, lsem1, ssem0, ssem1):
    wid = lax.axis_index("s") * _NC + lax.axis_index("c")
    rows_per_w = w_hbm.shape[0] // _NW
    n = rows_per_w // _CHUNK_ROWS
    base = wid * rows_per_w
    bufs = (buf0, buf1)
    lsems = (lsem0, lsem1)
    ssems = (ssem0, ssem1)
    loads = [
        pltpu.make_async_copy(
            w_hbm.at[pl.ds(base + i * _CHUNK_ROWS, _CHUNK_ROWS), :],
            bufs[i % 2],
            lsems[i % 2],
        )
        for i in range(n)
    ]
    stores = [
        pltpu.make_async_copy(
            bufs[i % 2],
            o_hbm.at[0, pl.ds(base + i * _CHUNK_ROWS, _CHUNK_ROWS), :],
            ssems[i % 2],
        )
        for i in range(n)
    ]
    loads[0].start()
    for i in range(n):
        if i + 1 < n:
            if i >= 1:
                stores[i - 1].wait()
            loads[i + 1].start()
        loads[i].wait()
        stores[i].start()
    stores[n - 1].wait()
    if n >= 2:
        stores[n - 2].wait()


def kernel(x, emb_weight):
    seq_len = x.shape[1]
    dim = emb_weight.shape[1]
    sc_copy = pl.kernel(
        _sc_copy_body,
        out_type=jax.ShapeDtypeStruct((1, seq_len, dim), emb_weight.dtype),
        mesh=plsc.VectorSubcoreMesh(core_axis_name="c", subcore_axis_name="s"),
        scratch_types=[
            pltpu.VMEM((_CHUNK_ROWS, dim), emb_weight.dtype),
            pltpu.VMEM((_CHUNK_ROWS, dim), emb_weight.dtype),
            pltpu.SemaphoreType.DMA,
            pltpu.SemaphoreType.DMA,
            pltpu.SemaphoreType.DMA,
            pltpu.SemaphoreType.DMA,
        ],
    )
    return sc_copy(emb_weight)
